# baseline (speedup 1.0000x reference)
import numpy as np
import jax
import jax.numpy as jnp
from jax.experimental import pallas as pl
from jax.experimental.pallas import tpu as pltpu

_B, _T, _D = 1, 2048, 768
_H, _HD, _TK = 12, 64, 32
_NL = 128
_VW = 128


def _topk_threshold(x):
    n = _NL
    li = jax.lax.broadcasted_iota(jnp.int32, (1, n), 1)
    s = x
    k = 2
    while k <= n:
        j = k // 2
        while j >= 1:
            p = jnp.take_along_axis(s, jnp.broadcast_to(li ^ j, s.shape), axis=1)
            keep_min = ((li & k) == 0) == ((li & j) == 0)
            s = jnp.where(keep_min, jnp.minimum(s, p), jnp.maximum(s, p))
            j //= 2
        k *= 2
    return jax.lax.slice_in_dim(s, n - _TK, n - _TK + 1, axis=1)


def _fused_kernel(lb_ref, x_ref, wq_ref, wk_ref, wv_ref, sink_ref, wo_ref,
                  out_ref):
    h = pl.program_id(0)
    x = x_ref[...]
    dims = (((1,), (1,)), ((), ()))
    q = jax.lax.dot_general(x, wq_ref[...], dims,
                            preferred_element_type=jnp.float32)
    q = jnp.maximum(q, 0.0)
    kk = jax.lax.dot_general(x, wk_ref[...], dims,
                             preferred_element_type=jnp.float32)
    kk = jnp.maximum(kk, 0.0)
    t = _topk_threshold(jnp.concatenate([q, kk], axis=0))
    qm = jnp.where(q >= t[:_T], q, 0.0).astype(jnp.bfloat16)
    km = jnp.where(kk >= t[_T:], kk, 0.0).astype(jnp.bfloat16)
    v = jax.lax.dot_general(x, wv_ref[...], dims,
                            preferred_element_type=jnp.float32)
    va = jnp.concatenate(
        [v, jnp.ones((_T, 1), jnp.float32),
         jnp.zeros((_T, _VW - _HD - 1), jnp.float32)], axis=1).astype(jnp.bfloat16)

    beta = jnp.exp(lb_ref[0, 0])
    s = jax.lax.dot_general(qm, km, (((1,), (1,)), ((), ())),
                            preferred_element_type=jnp.float32)
    r_iota = jax.lax.broadcasted_iota(jnp.int32, (_T, _T), 0)
    c_iota = jax.lax.broadcasted_iota(jnp.int32, (_T, _T), 1)
    s = jnp.where(r_iota >= c_iota, s, 0.0).astype(jnp.bfloat16)
    yd = jax.lax.dot_general(s, va, (((1,), (0,)), ((), ())),
                             preferred_element_type=jnp.float32)
    y = yd[:, :_HD]
    denom = yd[:, _HD:_HD + 1]
    dws = denom + beta
    y = y / jnp.maximum(dws, 1e-12) + (beta / dws) * sink_ref[pl.ds(h, 1), :]
    o = jax.lax.dot_general(y.astype(jnp.bfloat16), wo_ref[...],
                            (((1,), (0,)), ((), ())),
                            preferred_element_type=jnp.float32)

    @pl.when(h == 0)
    def _():
        out_ref[...] = o

    @pl.when(h > 0)
    def _():
        out_ref[...] += o


@jax.jit
def _run(x2, W_q, W_k, W_v, W_o, sink, log_beta):
    return pl.pallas_call(
        _fused_kernel,
        grid=(_H,),
        in_specs=[
            pl.BlockSpec((1, 1), lambda h: (0, 0), memory_space=pltpu.SMEM),
            pl.BlockSpec((_T, _D), lambda h: (0, 0)),
            pl.BlockSpec((_NL, _D), lambda h: (h, 0)),
            pl.BlockSpec((_NL, _D), lambda h: (h, 0)),
            pl.BlockSpec((_HD, _D), lambda h: (h, 0)),
            pl.BlockSpec((_H, _HD), lambda h: (0, 0)),
            pl.BlockSpec((_HD, _D), lambda h: (h, 0)),
        ],
        out_specs=pl.BlockSpec((_T, _D), lambda h: (0, 0)),
        out_shape=jax.ShapeDtypeStruct((_T, _D), jnp.float32),
    )(log_beta.reshape(1, 1), x2, W_q, W_k, W_v, sink,
      W_o.T.astype(jnp.bfloat16))


def kernel(x, W_q, W_k, W_v, W_o, sink, log_beta):
    out = _run(x.reshape(_T, _D), W_q, W_k, W_v, W_o, sink, log_beta)
    return out.reshape(_B, _T, _D)


# speedup vs baseline: 1.0749x; 1.0749x over previous
import functools

import numpy as np
import jax
import jax.numpy as jnp
from jax.experimental import pallas as pl
from jax.experimental.pallas import tpu as pltpu

_B, _T, _D = 1, 2048, 768
_H, _HD, _TK = 12, 64, 32
_NL = 128
_BT = 1024
_NI = _T // _BT
_VW = 128


def _topk_threshold(x):
    n = _NL
    lanes = np.arange(n)
    li = jax.lax.broadcasted_iota(jnp.int32, (1, n), 1)
    s = x
    k = 2
    while k <= n:
        j = k // 2
        while j >= 1:
            p = jnp.take_along_axis(s, jnp.broadcast_to(li ^ j, s.shape), axis=1)
            keep_min = ((li & k) == 0) == ((li & j) == 0)
            s = jnp.where(keep_min, jnp.minimum(s, p), jnp.maximum(s, p))
            j //= 2
        k *= 2
    return jax.lax.slice_in_dim(s, n - _TK, n - _TK + 1, axis=1)


def _proj_kernel(x_ref, wq_ref, wk_ref, wv_ref, qm_ref, km_ref, va_ref):
    x = x_ref[...]
    dims = (((1,), (1,)), ((), ()))
    q = jax.lax.dot_general(x, wq_ref[...], dims,
                            preferred_element_type=jnp.float32)
    q = jnp.maximum(q, 0.0)
    kk = jax.lax.dot_general(x, wk_ref[...], dims,
                             preferred_element_type=jnp.float32)
    kk = jnp.maximum(kk, 0.0)
    t = _topk_threshold(jnp.concatenate([q, kk], axis=0))
    qm_ref[...] = jnp.where(q >= t[:_BT], q, 0.0).astype(jnp.bfloat16)
    km_ref[...] = jnp.where(kk >= t[_BT:], kk, 0.0).astype(jnp.bfloat16)
    v = jax.lax.dot_general(x, wv_ref[...], dims,
                            preferred_element_type=jnp.float32)
    va_ref[...] = jnp.concatenate(
        [v, jnp.ones((_BT, 1), jnp.float32),
         jnp.zeros((_BT, _VW - _HD - 1), jnp.float32)], axis=1).astype(jnp.bfloat16)


def _attn_kernel(lb_ref, qm_ref, km_ref, va_ref, sink_ref, wo_ref, out_ref):
    i = pl.program_id(0)
    h = pl.program_id(1)
    q = qm_ref[...]
    beta = jnp.exp(lb_ref[0, 0])
    s = jax.lax.dot_general(q, km_ref[...], (((1,), (1,)), ((), ())),
                            preferred_element_type=jnp.float32)
    r_iota = jax.lax.broadcasted_iota(jnp.int32, (_BT, _T), 0)
    c_iota = jax.lax.broadcasted_iota(jnp.int32, (_BT, _T), 1)
    keep = (r_iota + i * _BT) >= c_iota
    s = jnp.where(keep, s, 0.0).astype(jnp.bfloat16)
    yd = jax.lax.dot_general(s, va_ref[...], (((1,), (0,)), ((), ())),
                             preferred_element_type=jnp.float32)
    y = yd[:, :_HD]
    denom = yd[:, _HD:_HD + 1]
    dws = denom + beta
    y = y / jnp.maximum(dws, 1e-12) + (beta / dws) * sink_ref[pl.ds(h, 1), :]
    o = jax.lax.dot_general(y.astype(jnp.bfloat16), wo_ref[...],
                            (((1,), (0,)), ((), ())),
                            preferred_element_type=jnp.float32)

    @pl.when(h == 0)
    def _():
        out_ref[...] = o

    @pl.when(h > 0)
    def _():
        out_ref[...] += o


@jax.jit
def _run(x2, W_q, W_k, W_v, W_o, sink, log_beta):
    qm, km, va = pl.pallas_call(
        _proj_kernel,
        grid=(_H, _NI),
        in_specs=[
            pl.BlockSpec((_BT, _D), lambda h, i: (i, 0)),
            pl.BlockSpec((_NL, _D), lambda h, i: (h, 0)),
            pl.BlockSpec((_NL, _D), lambda h, i: (h, 0)),
            pl.BlockSpec((_HD, _D), lambda h, i: (h, 0)),
        ],
        out_specs=[
            pl.BlockSpec((_BT, _NL), lambda h, i: (i, h)),
            pl.BlockSpec((_BT, _NL), lambda h, i: (i, h)),
            pl.BlockSpec((_BT, _VW), lambda h, i: (i, h)),
        ],
        out_shape=[
            jax.ShapeDtypeStruct((_T, _H * _NL), jnp.bfloat16),
            jax.ShapeDtypeStruct((_T, _H * _NL), jnp.bfloat16),
            jax.ShapeDtypeStruct((_T, _H * _VW), jnp.bfloat16),
        ],
        compiler_params=pltpu.CompilerParams(
            dimension_semantics=("parallel", "parallel")),
    )(x2, W_q, W_k, W_v)

    out = pl.pallas_call(
        _attn_kernel,
        grid=(_NI, _H),
        in_specs=[
            pl.BlockSpec((1, 1), lambda i, h: (0, 0), memory_space=pltpu.SMEM),
            pl.BlockSpec((_BT, _NL), lambda i, h: (i, h)),
            pl.BlockSpec((_T, _NL), lambda i, h: (0, h)),
            pl.BlockSpec((_T, _VW), lambda i, h: (0, h)),
            pl.BlockSpec((_H, _HD), lambda i, h: (0, 0)),
            pl.BlockSpec((_HD, _D), lambda i, h: (h, 0)),
        ],
        out_specs=pl.BlockSpec((_BT, _D), lambda i, h: (i, 0)),
        out_shape=jax.ShapeDtypeStruct((_T, _D), jnp.float32),
        compiler_params=pltpu.CompilerParams(
            dimension_semantics=("parallel", "arbitrary")),
    )(log_beta.reshape(1, 1), qm, km, va, sink,
      W_o.T.astype(jnp.bfloat16))
    return out


def kernel(x, W_q, W_k, W_v, W_o, sink, log_beta):
    out = _run(x.reshape(_T, _D), W_q, W_k, W_v, W_o, sink, log_beta)
    return out.reshape(_B, _T, _D)


# speedup vs baseline: 1.1194x; 1.0415x over previous
import functools

import numpy as np
import jax
import jax.numpy as jnp
from jax.experimental import pallas as pl
from jax.experimental.pallas import tpu as pltpu

_B, _T, _D = 1, 2048, 768
_H, _HD, _TK = 12, 64, 32
_NL = 128
_BT = 2048
_NI = _T // _BT
_VW = 128


def _topk_threshold(x):
    n = _NL
    lanes = np.arange(n)
    li = jax.lax.broadcasted_iota(jnp.int32, (1, n), 1)
    s = x
    k = 2
    while k <= n:
        j = k // 2
        while j >= 1:
            p = jnp.take_along_axis(s, jnp.broadcast_to(li ^ j, s.shape), axis=1)
            keep_min = ((li & k) == 0) == ((li & j) == 0)
            s = jnp.where(keep_min, jnp.minimum(s, p), jnp.maximum(s, p))
            j //= 2
        k *= 2
    return jax.lax.slice_in_dim(s, n - _TK, n - _TK + 1, axis=1)


def _proj_kernel(x_ref, wq_ref, wk_ref, wv_ref, qm_ref, km_ref, va_ref):
    x = x_ref[...]
    dims = (((1,), (1,)), ((), ()))
    q = jax.lax.dot_general(x, wq_ref[...], dims,
                            preferred_element_type=jnp.float32)
    q = jnp.maximum(q, 0.0)
    kk = jax.lax.dot_general(x, wk_ref[...], dims,
                             preferred_element_type=jnp.float32)
    kk = jnp.maximum(kk, 0.0)
    t = _topk_threshold(jnp.concatenate([q, kk], axis=0))
    qm_ref[...] = jnp.where(q >= t[:_BT], q, 0.0).astype(jnp.bfloat16)
    km_ref[...] = jnp.where(kk >= t[_BT:], kk, 0.0).astype(jnp.bfloat16)
    v = jax.lax.dot_general(x, wv_ref[...], dims,
                            preferred_element_type=jnp.float32)
    va_ref[...] = jnp.concatenate(
        [v, jnp.ones((_BT, 1), jnp.float32),
         jnp.zeros((_BT, _VW - _HD - 1), jnp.float32)], axis=1).astype(jnp.bfloat16)


def _attn_kernel(lb_ref, qm_ref, km_ref, va_ref, sink_ref, wo_ref, out_ref,
                 mask_ref):
    i = pl.program_id(0)
    h = pl.program_id(1)

    @pl.when(h == 0)
    def _():
        r_iota = jax.lax.broadcasted_iota(jnp.int32, (_BT, _T), 0)
        c_iota = jax.lax.broadcasted_iota(jnp.int32, (_BT, _T), 1)
        mask_ref[...] = ((r_iota + i * _BT) >= c_iota).astype(jnp.bfloat16)

    q = qm_ref[...]
    beta = jnp.exp(lb_ref[0, 0])
    s = jax.lax.dot_general(q, km_ref[...], (((1,), (1,)), ((), ())),
                            preferred_element_type=jnp.float32)
    s = s.astype(jnp.bfloat16) * mask_ref[...]
    yd = jax.lax.dot_general(s, va_ref[...], (((1,), (0,)), ((), ())),
                             preferred_element_type=jnp.float32)
    y = yd[:, :_HD]
    denom = yd[:, _HD:_HD + 1]
    dws = denom + beta
    y = y / jnp.maximum(dws, 1e-12) + (beta / dws) * sink_ref[pl.ds(h, 1), :]
    o = jax.lax.dot_general(y.astype(jnp.bfloat16), wo_ref[...],
                            (((1,), (0,)), ((), ())),
                            preferred_element_type=jnp.float32)

    @pl.when(h == 0)
    def _():
        out_ref[...] = o

    @pl.when(h > 0)
    def _():
        out_ref[...] += o


@jax.jit
def _run(x2, W_q, W_k, W_v, W_o, sink, log_beta):
    qm, km, va = pl.pallas_call(
        _proj_kernel,
        grid=(_H, _NI),
        in_specs=[
            pl.BlockSpec((_BT, _D), lambda h, i: (i, 0)),
            pl.BlockSpec((_NL, _D), lambda h, i: (h, 0)),
            pl.BlockSpec((_NL, _D), lambda h, i: (h, 0)),
            pl.BlockSpec((_HD, _D), lambda h, i: (h, 0)),
        ],
        out_specs=[
            pl.BlockSpec((_BT, _NL), lambda h, i: (i, h)),
            pl.BlockSpec((_BT, _NL), lambda h, i: (i, h)),
            pl.BlockSpec((_BT, _VW), lambda h, i: (i, h)),
        ],
        out_shape=[
            jax.ShapeDtypeStruct((_T, _H * _NL), jnp.bfloat16),
            jax.ShapeDtypeStruct((_T, _H * _NL), jnp.bfloat16),
            jax.ShapeDtypeStruct((_T, _H * _VW), jnp.bfloat16),
        ],
    )(x2, W_q, W_k, W_v)

    out = pl.pallas_call(
        _attn_kernel,
        grid=(_NI, _H),
        in_specs=[
            pl.BlockSpec((1, 1), lambda i, h: (0, 0), memory_space=pltpu.SMEM),
            pl.BlockSpec((_BT, _NL), lambda i, h: (i, h)),
            pl.BlockSpec((_T, _NL), lambda i, h: (0, h)),
            pl.BlockSpec((_T, _VW), lambda i, h: (0, h)),
            pl.BlockSpec((_H, _HD), lambda i, h: (0, 0)),
            pl.BlockSpec((_HD, _D), lambda i, h: (h, 0)),
        ],
        out_specs=pl.BlockSpec((_BT, _D), lambda i, h: (i, 0)),
        out_shape=jax.ShapeDtypeStruct((_T, _D), jnp.float32),
        scratch_shapes=[pltpu.VMEM((_BT, _T), jnp.bfloat16)],
    )(log_beta.reshape(1, 1), qm, km, va, sink,
      W_o.T.astype(jnp.bfloat16))
    return out


def kernel(x, W_q, W_k, W_v, W_o, sink, log_beta):
    out = _run(x.reshape(_T, _D), W_q, W_k, W_v, W_o, sink, log_beta)
    return out.reshape(_B, _T, _D)


# speedup vs baseline: 1.2079x; 1.0790x over previous
import functools

import numpy as np
import jax
import jax.numpy as jnp
from jax.experimental import pallas as pl
from jax.experimental.pallas import tpu as pltpu

_B, _T, _D = 1, 2048, 768
_H, _HD, _TK = 12, 64, 32
_NL = 128
_BT = 2048
_NI = _T // _BT
_VW = 128


def _topk_threshold(x):
    n = _NL
    lanes = np.arange(n)
    li = jax.lax.broadcasted_iota(jnp.int32, (1, n), 1)
    s = x
    k = 2
    while k <= n:
        j = k // 2
        while j >= 1:
            p = jnp.take_along_axis(s, jnp.broadcast_to(li ^ j, s.shape), axis=1)
            keep_min = ((li & k) == 0) == ((li & j) == 0)
            s = jnp.where(keep_min, jnp.minimum(s, p), jnp.maximum(s, p))
            j //= 2
        k *= 2
    return jax.lax.slice_in_dim(s, n - _TK, n - _TK + 1, axis=1)


def _proj_kernel(x_ref, wq_ref, wk_ref, wv_ref, qm_ref, km_ref, va_ref):
    x = x_ref[...]
    dims = (((1,), (1,)), ((), ()))
    q = jax.lax.dot_general(x, wq_ref[...], dims,
                            preferred_element_type=jnp.float32)
    q = jnp.maximum(q, 0.0)
    kk = jax.lax.dot_general(x, wk_ref[...], dims,
                             preferred_element_type=jnp.float32)
    kk = jnp.maximum(kk, 0.0)
    t = _topk_threshold(jnp.concatenate([q, kk], axis=0))
    qm_ref[...] = jnp.where(q >= t[:_BT], q, 0.0).astype(jnp.bfloat16)
    km_ref[...] = jnp.where(kk >= t[_BT:], kk, 0.0).astype(jnp.bfloat16)
    v = jax.lax.dot_general(x, wv_ref[...], dims,
                            preferred_element_type=jnp.float32)
    va_ref[...] = jnp.concatenate(
        [v, jnp.ones((_BT, 1), jnp.float32),
         jnp.zeros((_BT, _VW - _HD - 1), jnp.float32)], axis=1).astype(jnp.bfloat16)


def _attn_kernel(lb_ref, qm_ref, km_ref, va_ref, sink_ref, out_ref,
                 mask_ref):
    i = pl.program_id(0)
    h = pl.program_id(1)

    @pl.when(h == 0)
    def _():
        r_iota = jax.lax.broadcasted_iota(jnp.int32, (_BT, _T), 0)
        c_iota = jax.lax.broadcasted_iota(jnp.int32, (_BT, _T), 1)
        mask_ref[...] = ((r_iota + i * _BT) >= c_iota).astype(jnp.bfloat16)

    q = qm_ref[...]
    beta = jnp.exp(lb_ref[0, 0])
    s = jax.lax.dot_general(q, km_ref[...], (((1,), (1,)), ((), ())),
                            preferred_element_type=jnp.float32)
    s = s.astype(jnp.bfloat16) * mask_ref[...]
    yd = jax.lax.dot_general(s, va_ref[...], (((1,), (0,)), ((), ())),
                             preferred_element_type=jnp.float32)
    y = yd[:, :_HD]
    denom = yd[:, _HD:_HD + 1]
    dws = denom + beta
    y = y / jnp.maximum(dws, 1e-12) + (beta / dws) * sink_ref[pl.ds(h, 1), :]
    out_ref[...] = y.astype(jnp.bfloat16).T


def _oproj_kernel(yb_ref, wo_ref, out_ref):
    out_ref[...] = jax.lax.dot_general(
        yb_ref[...], wo_ref[...], (((0,), (1,)), ((), ())),
        preferred_element_type=jnp.float32)


@jax.jit
def _run(x2, W_q, W_k, W_v, W_o, sink, log_beta):
    qm, km, va = pl.pallas_call(
        _proj_kernel,
        grid=(_H, _NI),
        in_specs=[
            pl.BlockSpec((_BT, _D), lambda h, i: (i, 0)),
            pl.BlockSpec((_NL, _D), lambda h, i: (h, 0)),
            pl.BlockSpec((_NL, _D), lambda h, i: (h, 0)),
            pl.BlockSpec((_HD, _D), lambda h, i: (h, 0)),
        ],
        out_specs=[
            pl.BlockSpec((_BT, _NL), lambda h, i: (i, h)),
            pl.BlockSpec((_BT, _NL), lambda h, i: (i, h)),
            pl.BlockSpec((_BT, _VW), lambda h, i: (i, h)),
        ],
        out_shape=[
            jax.ShapeDtypeStruct((_T, _H * _NL), jnp.bfloat16),
            jax.ShapeDtypeStruct((_T, _H * _NL), jnp.bfloat16),
            jax.ShapeDtypeStruct((_T, _H * _VW), jnp.bfloat16),
        ],
    )(x2, W_q, W_k, W_v)

    yb = pl.pallas_call(
        _attn_kernel,
        grid=(_NI, _H),
        in_specs=[
            pl.BlockSpec((1, 1), lambda i, h: (0, 0), memory_space=pltpu.SMEM),
            pl.BlockSpec((_BT, _NL), lambda i, h: (i, h)),
            pl.BlockSpec((_T, _NL), lambda i, h: (0, h)),
            pl.BlockSpec((_T, _VW), lambda i, h: (0, h)),
            pl.BlockSpec((_H, _HD), lambda i, h: (0, 0)),
        ],
        out_specs=pl.BlockSpec((_HD, _BT), lambda i, h: (h, i)),
        out_shape=jax.ShapeDtypeStruct((_H * _HD, _T), jnp.bfloat16),
        scratch_shapes=[pltpu.VMEM((_BT, _T), jnp.bfloat16)],
    )(log_beta.reshape(1, 1), qm, km, va, sink)

    out = pl.pallas_call(
        _oproj_kernel,
        grid=(1,),
        in_specs=[
            pl.BlockSpec((_H * _HD, _T), lambda g: (0, 0)),
            pl.BlockSpec((_D, _H * _HD), lambda g: (0, 0)),
        ],
        out_specs=pl.BlockSpec((_T, _D), lambda g: (0, 0)),
        out_shape=jax.ShapeDtypeStruct((_T, _D), jnp.float32),
    )(yb, W_o.astype(jnp.bfloat16))
    return out


def kernel(x, W_q, W_k, W_v, W_o, sink, log_beta):
    out = _run(x.reshape(_T, _D), W_q, W_k, W_v, W_o, sink, log_beta)
    return out.reshape(_B, _T, _D)


# speedup vs baseline: 1.2640x; 1.0464x over previous
import functools

import numpy as np
import jax
import jax.numpy as jnp
from jax.experimental import pallas as pl
from jax.experimental.pallas import tpu as pltpu

_B, _T, _D = 1, 2048, 768
_H, _HD, _TK = 12, 64, 32
_NL = 128
_BT = 2048
_NI = _T // _BT
_VW = 128


def _topk_threshold(x):
    n = _NL
    lanes = np.arange(n)
    li = jax.lax.broadcasted_iota(jnp.int32, (1, n), 1)
    s = x
    k = 2
    while k <= n:
        j = k // 2
        while j >= 1:
            p = jnp.take_along_axis(s, jnp.broadcast_to(li ^ j, s.shape), axis=1)
            keep_min = ((li & k) == 0) == ((li & j) == 0)
            s = jnp.where(keep_min, jnp.minimum(s, p), jnp.maximum(s, p))
            j //= 2
        k *= 2
    return jax.lax.slice_in_dim(s, n - _TK, n - _TK + 1, axis=1)


def _proj_kernel(x_ref, wq_ref, wk_ref, wv_ref, qm_ref, km_ref, va_ref):
    x = x_ref[...]
    dims = (((1,), (1,)), ((), ()))
    q = jax.lax.dot_general(x, wq_ref[...], dims,
                            preferred_element_type=jnp.float32)
    q = jnp.maximum(q, 0.0)
    kk = jax.lax.dot_general(x, wk_ref[...], dims,
                             preferred_element_type=jnp.float32)
    kk = jnp.maximum(kk, 0.0)
    qm_ref[...] = jnp.where(q >= _topk_threshold(q), q, 0.0).astype(jnp.bfloat16)
    km_ref[...] = jnp.where(kk >= _topk_threshold(kk), kk, 0.0).astype(jnp.bfloat16)
    v = jax.lax.dot_general(x, wv_ref[...], dims,
                            preferred_element_type=jnp.float32)
    va_ref[...] = jnp.concatenate(
        [v, jnp.ones((_BT, 1), jnp.float32),
         jnp.zeros((_BT, _VW - _HD - 1), jnp.float32)], axis=1).astype(jnp.bfloat16)


def _attn_kernel(lb_ref, qm_ref, km_ref, va_ref, sink_ref, out_ref,
                 mask_ref):
    i = pl.program_id(0)
    h = pl.program_id(1)

    @pl.when(h == 0)
    def _():
        r_iota = jax.lax.broadcasted_iota(jnp.int32, (_BT, _T), 0)
        c_iota = jax.lax.broadcasted_iota(jnp.int32, (_BT, _T), 1)
        mask_ref[...] = ((r_iota + i * _BT) >= c_iota).astype(jnp.bfloat16)

    q = qm_ref[...]
    beta = jnp.exp(lb_ref[0, 0])
    s = jax.lax.dot_general(q, km_ref[...], (((1,), (1,)), ((), ())),
                            preferred_element_type=jnp.float32)
    s = s.astype(jnp.bfloat16) * mask_ref[...]
    yd = jax.lax.dot_general(s, va_ref[...], (((1,), (0,)), ((), ())),
                             preferred_element_type=jnp.float32)
    y = yd[:, :_HD]
    denom = yd[:, _HD:_HD + 1]
    dws = denom + beta
    y = y / jnp.maximum(dws, 1e-12) + (beta / dws) * sink_ref[pl.ds(h, 1), :]
    out_ref[...] = y.astype(jnp.bfloat16).T


def _oproj_kernel(yb_ref, wo_ref, out_ref):
    out_ref[...] = jax.lax.dot_general(
        yb_ref[...], wo_ref[...], (((0,), (1,)), ((), ())),
        preferred_element_type=jnp.float32)


@jax.jit
def _run(x2, W_q, W_k, W_v, W_o, sink, log_beta):
    qm, km, va = pl.pallas_call(
        _proj_kernel,
        grid=(_H, _NI),
        in_specs=[
            pl.BlockSpec((_BT, _D), lambda h, i: (i, 0)),
            pl.BlockSpec((_NL, _D), lambda h, i: (h, 0)),
            pl.BlockSpec((_NL, _D), lambda h, i: (h, 0)),
            pl.BlockSpec((_HD, _D), lambda h, i: (h, 0)),
        ],
        out_specs=[
            pl.BlockSpec((_BT, _NL), lambda h, i: (i, h)),
            pl.BlockSpec((_BT, _NL), lambda h, i: (i, h)),
            pl.BlockSpec((_BT, _VW), lambda h, i: (i, h)),
        ],
        out_shape=[
            jax.ShapeDtypeStruct((_T, _H * _NL), jnp.bfloat16),
            jax.ShapeDtypeStruct((_T, _H * _NL), jnp.bfloat16),
            jax.ShapeDtypeStruct((_T, _H * _VW), jnp.bfloat16),
        ],
    )(x2, W_q, W_k, W_v)

    yb = pl.pallas_call(
        _attn_kernel,
        grid=(_NI, _H),
        in_specs=[
            pl.BlockSpec((1, 1), lambda i, h: (0, 0), memory_space=pltpu.SMEM),
            pl.BlockSpec((_BT, _NL), lambda i, h: (i, h)),
            pl.BlockSpec((_T, _NL), lambda i, h: (0, h)),
            pl.BlockSpec((_T, _VW), lambda i, h: (0, h)),
            pl.BlockSpec((_H, _HD), lambda i, h: (0, 0)),
        ],
        out_specs=pl.BlockSpec((_HD, _BT), lambda i, h: (h, i)),
        out_shape=jax.ShapeDtypeStruct((_H * _HD, _T), jnp.bfloat16),
        scratch_shapes=[pltpu.VMEM((_BT, _T), jnp.bfloat16)],
    )(log_beta.reshape(1, 1), qm, km, va, sink)

    out = pl.pallas_call(
        _oproj_kernel,
        grid=(1,),
        in_specs=[
            pl.BlockSpec((_H * _HD, _T), lambda g: (0, 0)),
            pl.BlockSpec((_D, _H * _HD), lambda g: (0, 0)),
        ],
        out_specs=pl.BlockSpec((_T, _D), lambda g: (0, 0)),
        out_shape=jax.ShapeDtypeStruct((_T, _D), jnp.float32),
    )(yb, W_o.astype(jnp.bfloat16))
    return out


def kernel(x, W_q, W_k, W_v, W_o, sink, log_beta):
    out = _run(x.reshape(_T, _D), W_q, W_k, W_v, W_o, sink, log_beta)
    return out.reshape(_B, _T, _D)
